# Initial kernel scaffold; baseline (speedup 1.0000x reference)
import jax
import jax.numpy as jnp
from jax.experimental import pallas as pl


def kernel(x, w1, b1, w2, b2):
    raise NotImplementedError("write your pallas kernel here")


# speedup vs baseline: 14.8813x; 14.8813x over previous
import jax
import jax.numpy as jnp
from jax.experimental import pallas as pl
from jax.experimental.pallas import tpu as pltpu

C = 4
H = 16
W = 16
LANES = H * W
N_STEPS = 8
OFF = 128


def _tap_masks_1d():
    idx = jnp.arange(LANES)
    i, j = idx // W, idx % W
    ms = []
    for kh in range(3):
        for kw in range(3):
            valid = ((i + (kh - 1) >= 0) & (i + (kh - 1) < H) &
                     (j + (kw - 1) >= 0) & (j + (kw - 1) < W))
            ms.append(valid.astype(jnp.float32))
    return jnp.stack(ms, axis=0)


def _blockdiag_taps(w):
    wp = jnp.zeros((9, 8, 8), jnp.float32)
    for kh in range(3):
        for kw in range(3):
            tap = kh * 3 + kw
            w4 = w[:, :, kh, kw]
            wp = wp.at[tap, :C, :C].set(w4)
            wp = wp.at[tap, C:, C:].set(w4)
    return wp


def _ode_kernel(x_ref, w1_ref, w2_ref, mask_ref, t1_ref, b1_ref, b2_ref,
                o_ref, pad_ref):
    nbl = x_ref.shape[1]
    pad_ref[:, :OFF] = jnp.zeros((8, OFF), jnp.float32)
    pad_ref[:, OFF + nbl:] = jnp.zeros((8, OFF), jnp.float32)

    def conv3x3(slab, w_ref, acc):
        pad_ref[:, OFF:OFF + nbl] = slab
        for kh in range(3):
            for kw in range(3):
                tap = kh * 3 + kw
                d = (kh - 1) * W + (kw - 1)
                shifted = pad_ref[:, OFF + d:OFF + d + nbl]
                masked = shifted * mask_ref[tap]
                acc = acc + jnp.dot(w_ref[tap], masked,
                                    preferred_element_type=jnp.float32)
        return acc

    def odefunc(t, y):
        h = conv3x3(y, w1_ref, t * t1_ref[...] + b1_ref[...])
        h = jnp.maximum(h, 0.0)
        return conv3x3(h, w2_ref, b2_ref[...])

    dt = jnp.float32(1.0 / N_STEPS)

    def rk4_step(i, y):
        t = i.astype(jnp.float32) * dt
        k1 = odefunc(t, y)
        acc = y + (dt / 6.0) * k1
        k2 = odefunc(t + 0.5 * dt, y + (0.5 * dt) * k1)
        acc = acc + (dt / 3.0) * k2
        k3 = odefunc(t + 0.5 * dt, y + (0.5 * dt) * k2)
        acc = acc + (dt / 3.0) * k3
        k4 = odefunc(t + dt, y + dt * k3)
        return acc + (dt / 6.0) * k4

    o_ref[...] = jax.lax.fori_loop(0, N_STEPS, rk4_step, x_ref[...])


def kernel(x, w1, b1, w2, b2):
    b = x.shape[0]
    bh = b // 2
    nl = 64 if bh % 64 == 0 else bh
    nbl = nl * LANES
    np_ = bh // nl

    x = x.astype(jnp.float32)
    xp = (x.reshape(2, bh, C, LANES)
           .transpose(0, 2, 1, 3)
           .reshape(2 * C, bh * LANES))

    m9 = _tap_masks_1d()
    mask = jnp.broadcast_to(
        jnp.tile(m9, (1, nl))[:, None, :], (9, 8, nbl))

    t1 = (w1[:, 0, :, :].reshape(C, 9).astype(jnp.float32) @ m9)
    t1 = jnp.tile(jnp.concatenate([t1, t1], axis=0), (1, nl))
    b1f = jnp.broadcast_to(jnp.tile(b1.astype(jnp.float32), 2)[:, None],
                           (8, nbl))
    b2f = jnp.broadcast_to(jnp.tile(b2.astype(jnp.float32), 2)[:, None],
                           (8, nbl))

    w1p = _blockdiag_taps(w1[:, 1:, :, :].astype(jnp.float32))
    w2p = _blockdiag_taps(w2.astype(jnp.float32))

    out = pl.pallas_call(
        _ode_kernel,
        out_shape=jax.ShapeDtypeStruct((2 * C, bh * LANES), jnp.float32),
        grid=(np_,),
        in_specs=[
            pl.BlockSpec((2 * C, nbl), lambda p: (0, p)),
            pl.BlockSpec((9, 8, 8), lambda p: (0, 0, 0)),
            pl.BlockSpec((9, 8, 8), lambda p: (0, 0, 0)),
            pl.BlockSpec((9, 8, nbl), lambda p: (0, 0, 0)),
            pl.BlockSpec((8, nbl), lambda p: (0, 0)),
            pl.BlockSpec((8, nbl), lambda p: (0, 0)),
            pl.BlockSpec((8, nbl), lambda p: (0, 0)),
        ],
        out_specs=pl.BlockSpec((2 * C, nbl), lambda p: (0, p)),
        scratch_shapes=[pltpu.VMEM((8, nbl + 2 * OFF), jnp.float32)],
        compiler_params=pltpu.CompilerParams(
            dimension_semantics=("parallel",)),
    )(xp, w1p, w2p, mask, t1, b1f, b2f)

    return (out.reshape(2, C, bh, LANES)
               .transpose(0, 2, 1, 3)
               .reshape(b, C, H, W))


# speedup vs baseline: 51.5035x; 3.4609x over previous
import jax
import jax.numpy as jnp
from jax.experimental import pallas as pl
from jax.experimental.pallas import tpu as pltpu

C = 4
H = 16
W = 16
N_STEPS = 8
R = C * H
OFF = 128


def _band_weights(w):
    w = w.astype(jnp.float32)
    ms = []
    for s in (-1, 0, 1):
        m = jnp.zeros((R, R), jnp.float32)
        for kh in range(3):
            eye = jnp.eye(H, H, k=kh - 1, dtype=jnp.float32)
            m = m + jnp.kron(w[:, :, kh, s + 1], eye)
        ms.append(m)
    return jnp.stack(ms, axis=0)


def _ode_kernel(x_ref, m1_ref, m2_ref, ml_ref, mr_ref, t1_ref, b1_ref,
                b2_ref, o_ref, pad_ref):
    nl = x_ref.shape[1]
    pad_ref[:, :OFF] = jnp.zeros((R, OFF), jnp.float32)
    pad_ref[:, OFF + nl:] = jnp.zeros((R, OFF), jnp.float32)

    def conv3x3(slab, m_ref, acc):
        pad_ref[:, OFF:OFF + nl] = slab
        left = pad_ref[:, OFF - 1:OFF - 1 + nl] * ml_ref[...]
        right = pad_ref[:, OFF + 1:OFF + 1 + nl] * mr_ref[...]
        acc = acc + jnp.dot(m_ref[0], left,
                            preferred_element_type=jnp.float32)
        acc = acc + jnp.dot(m_ref[1], slab,
                            preferred_element_type=jnp.float32)
        acc = acc + jnp.dot(m_ref[2], right,
                            preferred_element_type=jnp.float32)
        return acc

    def odefunc(t, y):
        h = conv3x3(y, m1_ref, t * t1_ref[...] + b1_ref[...])
        h = jnp.maximum(h, 0.0)
        return conv3x3(h, m2_ref, b2_ref[...])

    dt = jnp.float32(1.0 / N_STEPS)

    def rk4_step(i, y):
        t = i.astype(jnp.float32) * dt
        k1 = odefunc(t, y)
        acc = y + (dt / 6.0) * k1
        k2 = odefunc(t + 0.5 * dt, y + (0.5 * dt) * k1)
        acc = acc + (dt / 3.0) * k2
        k3 = odefunc(t + 0.5 * dt, y + (0.5 * dt) * k2)
        acc = acc + (dt / 3.0) * k3
        k4 = odefunc(t + dt, y + dt * k3)
        return acc + (dt / 6.0) * k4

    o_ref[...] = jax.lax.fori_loop(0, N_STEPS, rk4_step, x_ref[...])


def kernel(x, w1, b1, w2, b2):
    b = x.shape[0]
    nb = 256 if b % 256 == 0 else b
    nl = nb * W
    np_ = b // nb

    x = x.astype(jnp.float32)
    bb = b // 8
    xp = (x.reshape(bb, 8, C, H, W)
           .transpose(2, 3, 0, 1, 4)
           .reshape(R, b * W))

    wv = jnp.arange(W)
    ml_pat = (wv >= 1).astype(jnp.float32)
    mr_pat = (wv <= W - 2).astype(jnp.float32)
    ml = jnp.broadcast_to(jnp.tile(ml_pat, (nb,))[None, :], (R, nl))
    mr = jnp.broadcast_to(jnp.tile(mr_pat, (nb,))[None, :], (R, nl))

    hv = jnp.arange(H)
    vh = jnp.stack([((hv + k - 1) >= 0) & ((hv + k - 1) < H)
                    for k in range(3)]).astype(jnp.float32)
    vw = jnp.stack([((wv + k - 1) >= 0) & ((wv + k - 1) < W)
                    for k in range(3)]).astype(jnp.float32)
    t1 = jnp.einsum('ckl,kh,lw->chw', w1[:, 0].astype(jnp.float32), vh, vw)
    t1 = jnp.tile(t1.reshape(R, W), (1, nb))

    b1f = jnp.broadcast_to(
        jnp.repeat(b1.astype(jnp.float32), H)[:, None], (R, nl))
    b2f = jnp.broadcast_to(
        jnp.repeat(b2.astype(jnp.float32), H)[:, None], (R, nl))

    m1 = _band_weights(w1[:, 1:])
    m2 = _band_weights(w2)

    out = pl.pallas_call(
        _ode_kernel,
        out_shape=jax.ShapeDtypeStruct((R, b * W), jnp.float32),
        grid=(np_,),
        in_specs=[
            pl.BlockSpec((R, nl), lambda p: (0, p)),
            pl.BlockSpec((3, R, R), lambda p: (0, 0, 0)),
            pl.BlockSpec((3, R, R), lambda p: (0, 0, 0)),
            pl.BlockSpec((R, nl), lambda p: (0, 0)),
            pl.BlockSpec((R, nl), lambda p: (0, 0)),
            pl.BlockSpec((R, nl), lambda p: (0, 0)),
            pl.BlockSpec((R, nl), lambda p: (0, 0)),
            pl.BlockSpec((R, nl), lambda p: (0, 0)),
        ],
        out_specs=pl.BlockSpec((R, nl), lambda p: (0, p)),
        scratch_shapes=[pltpu.VMEM((R, nl + 2 * OFF), jnp.float32)],
        compiler_params=pltpu.CompilerParams(
            dimension_semantics=("parallel",)),
    )(xp, m1, m2, ml, mr, t1, b1f, b2f)

    return (out.reshape(C, H, bb, 8, W)
               .transpose(2, 3, 0, 1, 4)
               .reshape(b, C, H, W))


# speedup vs baseline: 51.5849x; 1.0016x over previous
import jax
import jax.numpy as jnp
from jax.experimental import pallas as pl
from jax.experimental.pallas import tpu as pltpu

C = 4
H = 16
W = 16
N_STEPS = 8
R = C * H
RA = R + 8
OFF = 128


def _band_weights(w):
    w = w.astype(jnp.float32)
    ms = []
    for s in (-1, 0, 1):
        m = jnp.zeros((R, R), jnp.float32)
        for kh in range(3):
            eye = jnp.eye(H, H, k=kh - 1, dtype=jnp.float32)
            m = m + jnp.kron(w[:, :, kh, s + 1], eye)
        ms.append(m)
    return ms


def _vh():
    hv = jnp.arange(H)
    return jnp.stack([((hv + k - 1) >= 0) & ((hv + k - 1) < H)
                      for k in range(3)]).astype(jnp.float32)


def _ode_kernel(x_ref, m1_ref, m2_ref, ml_ref, mr_ref, tw_ref, o_ref,
                pad_ref):
    nl = x_ref.shape[1]
    pad_ref[:, :OFF] = jnp.zeros((RA, OFF), jnp.float32)
    pad_ref[:, OFF + nl:] = jnp.zeros((RA, OFF), jnp.float32)
    pad_ref[R + 3:, OFF:OFF + nl] = jnp.concatenate(
        [tw_ref[3:4], jnp.zeros((4, nl), jnp.float32)], axis=0)

    def conv3x3(slab, m_ref):
        pad_ref[:R, OFF:OFF + nl] = slab
        left = pad_ref[:R, OFF - 1:OFF - 1 + nl] * ml_ref[...]
        right = pad_ref[:R, OFF + 1:OFF + 1 + nl] * mr_ref[...]
        acc = jnp.dot(m_ref[1], pad_ref[:, OFF:OFF + nl],
                      preferred_element_type=jnp.float32)
        acc = acc + jnp.dot(m_ref[0][:, :R], left,
                            preferred_element_type=jnp.float32)
        acc = acc + jnp.dot(m_ref[2][:, :R], right,
                            preferred_element_type=jnp.float32)
        return acc

    def odefunc(t, y):
        pad_ref[R:R + 3, OFF:OFF + nl] = t * tw_ref[:3]
        h = jnp.maximum(conv3x3(y, m1_ref), 0.0)
        return conv3x3(h, m2_ref)

    dt = jnp.float32(1.0 / N_STEPS)

    def rk4_step(i, y):
        t = i.astype(jnp.float32) * dt
        k1 = odefunc(t, y)
        acc = y + (dt / 6.0) * k1
        k2 = odefunc(t + 0.5 * dt, y + (0.5 * dt) * k1)
        acc = acc + (dt / 3.0) * k2
        k3 = odefunc(t + 0.5 * dt, y + (0.5 * dt) * k2)
        acc = acc + (dt / 3.0) * k3
        k4 = odefunc(t + dt, y + dt * k3)
        return acc + (dt / 6.0) * k4

    o_ref[...] = jax.lax.fori_loop(0, N_STEPS, rk4_step, x_ref[...])


def kernel(x, w1, b1, w2, b2):
    b = x.shape[0]
    nb = 256 if b % 256 == 0 else b
    nl = nb * W
    np_ = b // nb

    x = x.astype(jnp.float32)
    bb = b // 8
    xp = (x.reshape(bb, 8, C, H, W)
           .transpose(2, 3, 0, 1, 4)
           .reshape(R, b * W))

    wv = jnp.arange(W)
    vw = jnp.stack([((wv + k - 1) >= 0) & ((wv + k - 1) < W)
                    for k in range(3)]).astype(jnp.float32)
    ml = jnp.broadcast_to(jnp.tile(vw[0], (nb,))[None, :], (R, nl))
    mr = jnp.broadcast_to(jnp.tile(vw[2], (nb,))[None, :], (R, nl))

    tw = jnp.concatenate(
        [jnp.tile(vw, (1, nb)),
         jnp.ones((1, nl), jnp.float32),
         jnp.zeros((4, nl), jnp.float32)], axis=0)

    vh = _vh()
    w1f = w1.astype(jnp.float32)
    m1t = jnp.einsum('ckl,kh->chl', w1f[:, 0], vh).reshape(R, 3)
    b1c = jnp.repeat(b1.astype(jnp.float32), H)[:, None]
    b2c = jnp.repeat(b2.astype(jnp.float32), H)[:, None]

    m1l, m1c, m1r = _band_weights(w1[:, 1:])
    m2l, m2c, m2r = _band_weights(w2)
    zpad = jnp.zeros((R, 4), jnp.float32)
    z8 = jnp.zeros((R, 8), jnp.float32)
    m1 = jnp.stack([jnp.concatenate([m1l, z8], axis=1),
                    jnp.concatenate([m1c, m1t, b1c, zpad], axis=1),
                    jnp.concatenate([m1r, z8], axis=1)], axis=0)
    m2 = jnp.stack([jnp.concatenate([m2l, z8], axis=1),
                    jnp.concatenate([m2c, jnp.zeros((R, 3), jnp.float32),
                                     b2c, zpad], axis=1),
                    jnp.concatenate([m2r, z8], axis=1)], axis=0)

    out = pl.pallas_call(
        _ode_kernel,
        out_shape=jax.ShapeDtypeStruct((R, b * W), jnp.float32),
        grid=(np_,),
        in_specs=[
            pl.BlockSpec((R, nl), lambda p: (0, p)),
            pl.BlockSpec((3, R, RA), lambda p: (0, 0, 0)),
            pl.BlockSpec((3, R, RA), lambda p: (0, 0, 0)),
            pl.BlockSpec((R, nl), lambda p: (0, 0)),
            pl.BlockSpec((R, nl), lambda p: (0, 0)),
            pl.BlockSpec((8, nl), lambda p: (0, 0)),
        ],
        out_specs=pl.BlockSpec((R, nl), lambda p: (0, p)),
        scratch_shapes=[pltpu.VMEM((RA, nl + 2 * OFF), jnp.float32)],
        compiler_params=pltpu.CompilerParams(
            dimension_semantics=("parallel",)),
    )(xp, m1, m2, ml, mr, tw)

    return (out.reshape(C, H, bb, 8, W)
               .transpose(2, 3, 0, 1, 4)
               .reshape(b, C, H, W))


# speedup vs baseline: 58.9857x; 1.1435x over previous
import jax
import jax.numpy as jnp
from jax.experimental import pallas as pl
from jax.experimental.pallas import tpu as pltpu

C = 4
H = 16
W = 16
N_STEPS = 8
R = C * H
RA = R + 8
RS = 208
OFF = 128


def _band_weights(w):
    w = w.astype(jnp.float32)
    ms = []
    for s in (-1, 0, 1):
        m = jnp.zeros((R, R), jnp.float32)
        for kh in range(3):
            eye = jnp.eye(H, H, k=kh - 1, dtype=jnp.float32)
            m = m + jnp.kron(w[:, :, kh, s + 1], eye)
        ms.append(m)
    return ms


def _ode_kernel(x_ref, m1_ref, m2_ref, ml_ref, mr_ref, tw_ref, o_ref,
                stk_ref):
    nl = x_ref.shape[1]
    stk_ref[:, :OFF] = jnp.zeros((RS, OFF), jnp.bfloat16)
    stk_ref[:, OFF + nl:] = jnp.zeros((RS, OFF), jnp.bfloat16)
    stk_ref[R + 3:RA, OFF:OFF + nl] = jnp.concatenate(
        [tw_ref[3:4], jnp.zeros((4, nl), jnp.bfloat16)], axis=0)
    stk_ref[200:, OFF:OFF + nl] = jnp.zeros((8, nl), jnp.bfloat16)

    def conv3x3(slab, m_ref):
        stk_ref[:R, OFF:OFF + nl] = slab.astype(jnp.bfloat16)
        stk_ref[RA:RA + R, OFF:OFF + nl] = (
            stk_ref[:R, OFF - 1:OFF - 1 + nl] * ml_ref[...])
        stk_ref[RA + R:200, OFF:OFF + nl] = (
            stk_ref[:R, OFF + 1:OFF + 1 + nl] * mr_ref[...])
        return jnp.dot(m_ref[...], stk_ref[:, OFF:OFF + nl],
                       preferred_element_type=jnp.float32)

    def odefunc(t, y):
        stk_ref[R:R + 3, OFF:OFF + nl] = t.astype(jnp.bfloat16) * tw_ref[:3]
        h = jnp.maximum(conv3x3(y, m1_ref), 0.0)
        return conv3x3(h, m2_ref)

    dt = jnp.float32(1.0 / N_STEPS)

    def rk4_step(i, y):
        t = i.astype(jnp.float32) * dt
        k1 = odefunc(t, y)
        acc = y + (dt / 6.0) * k1
        k2 = odefunc(t + 0.5 * dt, y + (0.5 * dt) * k1)
        acc = acc + (dt / 3.0) * k2
        k3 = odefunc(t + 0.5 * dt, y + (0.5 * dt) * k2)
        acc = acc + (dt / 3.0) * k3
        k4 = odefunc(t + dt, y + dt * k3)
        return acc + (dt / 6.0) * k4

    o_ref[...] = jax.lax.fori_loop(0, N_STEPS, rk4_step, x_ref[...])


def kernel(x, w1, b1, w2, b2):
    b = x.shape[0]
    nb = 512 if b % 512 == 0 else b
    nl = nb * W
    np_ = b // nb

    x = x.astype(jnp.float32)
    bb = b // 8
    xp = (x.reshape(bb, 8, C, H, W)
           .transpose(2, 3, 0, 1, 4)
           .reshape(R, b * W))

    wv = jnp.arange(W)
    vw = jnp.stack([((wv + k - 1) >= 0) & ((wv + k - 1) < W)
                    for k in range(3)]).astype(jnp.float32)
    ml = jnp.broadcast_to(
        jnp.tile(vw[0], (nb,))[None, :], (R, nl)).astype(jnp.bfloat16)
    mr = jnp.broadcast_to(
        jnp.tile(vw[2], (nb,))[None, :], (R, nl)).astype(jnp.bfloat16)

    tw = jnp.concatenate(
        [jnp.tile(vw, (1, nb)),
         jnp.ones((1, nl), jnp.float32),
         jnp.zeros((4, nl), jnp.float32)], axis=0).astype(jnp.bfloat16)

    hv = jnp.arange(H)
    vh = jnp.stack([((hv + k - 1) >= 0) & ((hv + k - 1) < H)
                    for k in range(3)]).astype(jnp.float32)
    m1t = jnp.einsum('ckl,kh->chl', w1[:, 0].astype(jnp.float32),
                     vh).reshape(R, 3)
    b1c = jnp.repeat(b1.astype(jnp.float32), H)[:, None]
    b2c = jnp.repeat(b2.astype(jnp.float32), H)[:, None]

    m1l, m1c, m1r = _band_weights(w1[:, 1:])
    m2l, m2c, m2r = _band_weights(w2)
    z4 = jnp.zeros((R, 4), jnp.float32)
    z3 = jnp.zeros((R, 3), jnp.float32)
    z8 = jnp.zeros((R, 8), jnp.float32)
    m1 = jnp.concatenate([m1c, m1t, b1c, z4, m1l, m1r, z8],
                         axis=1).astype(jnp.bfloat16)
    m2 = jnp.concatenate([m2c, z3, b2c, z4, m2l, m2r, z8],
                         axis=1).astype(jnp.bfloat16)

    out = pl.pallas_call(
        _ode_kernel,
        out_shape=jax.ShapeDtypeStruct((R, b * W), jnp.float32),
        grid=(np_,),
        in_specs=[
            pl.BlockSpec((R, nl), lambda p: (0, p)),
            pl.BlockSpec((R, RS), lambda p: (0, 0)),
            pl.BlockSpec((R, RS), lambda p: (0, 0)),
            pl.BlockSpec((R, nl), lambda p: (0, 0)),
            pl.BlockSpec((R, nl), lambda p: (0, 0)),
            pl.BlockSpec((8, nl), lambda p: (0, 0)),
        ],
        out_specs=pl.BlockSpec((R, nl), lambda p: (0, p)),
        scratch_shapes=[pltpu.VMEM((RS, nl + 2 * OFF), jnp.bfloat16)],
        compiler_params=pltpu.CompilerParams(
            dimension_semantics=("parallel",)),
    )(xp, m1, m2, ml, mr, tw)

    return (out.reshape(C, H, bb, 8, W)
               .transpose(2, 3, 0, 1, 4)
               .reshape(b, C, H, W))


# speedup vs baseline: 127.8699x; 2.1678x over previous
import jax
import jax.numpy as jnp
from jax.experimental import pallas as pl
from jax.experimental.pallas import tpu as pltpu

C = 4
H = 16
W = 16
N_STEPS = 8
R = C * H
RA = R + 8


def _band_weights(w):
    w = w.astype(jnp.float32)
    ms = []
    for s in (-1, 0, 1):
        m = jnp.zeros((R, R), jnp.float32)
        for kh in range(3):
            eye = jnp.eye(H, H, k=kh - 1, dtype=jnp.float32)
            m = m + jnp.kron(w[:, :, kh, s + 1], eye)
        ms.append(m)
    return ms


def _ode_kernel(x_ref, m1_ref, m2_ref, tw_ref, o_ref, pad_ref):
    nl = x_ref.shape[1]
    nb = nl // W
    pad_ref[:, :nb] = jnp.zeros((RA, nb), jnp.bfloat16)
    pad_ref[:, nb + nl:] = jnp.zeros((RA, nb), jnp.bfloat16)
    pad_ref[R + 3:, nb:nb + nl] = jnp.concatenate(
        [tw_ref[3:4], jnp.zeros((4, nl), jnp.bfloat16)], axis=0)

    def conv3x3(slab, m_ref):
        pad_ref[:R, nb:nb + nl] = slab.astype(jnp.bfloat16)
        acc = jnp.dot(m_ref[1], pad_ref[:, nb:nb + nl],
                      preferred_element_type=jnp.float32)
        acc = acc + jnp.dot(m_ref[0][:, :R], pad_ref[:R, :nl],
                            preferred_element_type=jnp.float32)
        acc = acc + jnp.dot(m_ref[2][:, :R], pad_ref[:R, 2 * nb:2 * nb + nl],
                            preferred_element_type=jnp.float32)
        return acc

    def odefunc(t, y):
        pad_ref[R:R + 3, nb:nb + nl] = t.astype(jnp.bfloat16) * tw_ref[:3]
        h = jnp.maximum(conv3x3(y, m1_ref), 0.0)
        return conv3x3(h, m2_ref)

    dt = jnp.float32(1.0 / N_STEPS)

    def rk4_step(i, y):
        t = i.astype(jnp.float32) * dt
        k1 = odefunc(t, y)
        acc = y + (dt / 6.0) * k1
        k2 = odefunc(t + 0.5 * dt, y + (0.5 * dt) * k1)
        acc = acc + (dt / 3.0) * k2
        k3 = odefunc(t + 0.5 * dt, y + (0.5 * dt) * k2)
        acc = acc + (dt / 3.0) * k3
        k4 = odefunc(t + dt, y + dt * k3)
        return acc + (dt / 6.0) * k4

    o_ref[...] = jax.lax.fori_loop(0, N_STEPS, rk4_step, x_ref[...])


def kernel(x, w1, b1, w2, b2):
    b = x.shape[0]
    nb = 512 if b % 512 == 0 else b
    nl = nb * W
    np_ = b // nb

    x = x.astype(jnp.float32)
    xp = (x.reshape(np_, nb, C, H, W)
           .transpose(2, 3, 0, 4, 1)
           .reshape(R, b * W))

    wv = jnp.arange(W)
    vw = jnp.stack([((wv + k - 1) >= 0) & ((wv + k - 1) < W)
                    for k in range(3)]).astype(jnp.float32)
    tw = jnp.concatenate(
        [jnp.repeat(vw, nb, axis=1),
         jnp.ones((1, nl), jnp.float32),
         jnp.zeros((4, nl), jnp.float32)], axis=0).astype(jnp.bfloat16)

    hv = jnp.arange(H)
    vh = jnp.stack([((hv + k - 1) >= 0) & ((hv + k - 1) < H)
                    for k in range(3)]).astype(jnp.float32)
    m1t = jnp.einsum('ckl,kh->chl', w1[:, 0].astype(jnp.float32),
                     vh).reshape(R, 3)
    b1c = jnp.repeat(b1.astype(jnp.float32), H)[:, None]
    b2c = jnp.repeat(b2.astype(jnp.float32), H)[:, None]

    m1l, m1c, m1r = _band_weights(w1[:, 1:])
    m2l, m2c, m2r = _band_weights(w2)
    z4 = jnp.zeros((R, 4), jnp.float32)
    z3 = jnp.zeros((R, 3), jnp.float32)
    z8 = jnp.zeros((R, 8), jnp.float32)
    m1 = jnp.stack([jnp.concatenate([m1l, z8], axis=1),
                    jnp.concatenate([m1c, m1t, b1c, z4], axis=1),
                    jnp.concatenate([m1r, z8], axis=1)],
                   axis=0).astype(jnp.bfloat16)
    m2 = jnp.stack([jnp.concatenate([m2l, z8], axis=1),
                    jnp.concatenate([m2c, z3, b2c, z4], axis=1),
                    jnp.concatenate([m2r, z8], axis=1)],
                   axis=0).astype(jnp.bfloat16)

    out = pl.pallas_call(
        _ode_kernel,
        out_shape=jax.ShapeDtypeStruct((R, b * W), jnp.float32),
        grid=(np_,),
        in_specs=[
            pl.BlockSpec((R, nl), lambda p: (0, p)),
            pl.BlockSpec((3, R, RA), lambda p: (0, 0, 0)),
            pl.BlockSpec((3, R, RA), lambda p: (0, 0, 0)),
            pl.BlockSpec((8, nl), lambda p: (0, 0)),
        ],
        out_specs=pl.BlockSpec((R, nl), lambda p: (0, p)),
        scratch_shapes=[pltpu.VMEM((RA, nl + 2 * (nl // W)), jnp.bfloat16)],
        compiler_params=pltpu.CompilerParams(
            dimension_semantics=("parallel",)),
    )(xp, m1, m2, tw)

    return (out.reshape(C, H, np_, W, nb)
               .transpose(2, 4, 0, 1, 3)
               .reshape(b, C, H, W))


# speedup vs baseline: 157.8121x; 1.2342x over previous
import jax
import jax.numpy as jnp
from jax.experimental import pallas as pl
from jax.experimental.pallas import tpu as pltpu

C = 4
H = 16
W = 16
N_STEPS = 8
R = C * H
RA = R + 8
RS = 208
NB = 256


def _band_weights(w):
    w = w.astype(jnp.float32)
    ms = []
    for s in (-1, 0, 1):
        m = jnp.zeros((R, R), jnp.float32)
        for kh in range(3):
            eye = jnp.eye(H, H, k=kh - 1, dtype=jnp.float32)
            m = m + jnp.kron(w[:, :, kh, s + 1], eye)
        ms.append(m)
    return ms


def _ode_kernel(x_ref, m1_ref, m2_ref, tw_ref, o_ref, stk_a, stk_b):
    nl = x_ref.shape[1]
    nl2 = nl // 2
    nb = nl2 // W
    for stk in (stk_a, stk_b):
        stk[:, :nb] = jnp.zeros((RS, nb), jnp.bfloat16)
        stk[:, nb + nl2:] = jnp.zeros((RS, nb), jnp.bfloat16)
        stk[R + 3:RA, nb:nb + nl2] = jnp.concatenate(
            [tw_ref[3:4], jnp.zeros((4, nl2), jnp.bfloat16)], axis=0)
        stk[RA:RA + R, nb:2 * nb] = jnp.zeros((R, nb), jnp.bfloat16)
        stk[RA + R:200, nl2:nb + nl2] = jnp.zeros((R, nb), jnp.bfloat16)
        stk[200:, nb:nb + nl2] = jnp.zeros((8, nl2), jnp.bfloat16)

    def conv3x3(slab16, m_ref, stk):
        stk[:R, nb:nb + nl2] = slab16
        stk[RA:RA + R, 2 * nb:2 * nb + nl2] = slab16
        stk[RA + R:200, :nl2] = slab16
        return jnp.dot(m_ref[...], stk[:, nb:nb + nl2],
                       preferred_element_type=jnp.float32)

    def odefunc(t, y16, stk):
        stk[R:R + 3, nb:nb + nl2] = t.astype(jnp.bfloat16) * tw_ref[:3]
        h = jnp.maximum(conv3x3(y16, m1_ref, stk).astype(jnp.bfloat16), 0)
        return conv3x3(h, m2_ref, stk)

    dt = jnp.float32(1.0 / N_STEPS)
    stks = (stk_a, stk_b)

    def rk4_step(i, ys):
        t = i.astype(jnp.float32) * dt
        k1 = [odefunc(t, ys[s].astype(jnp.bfloat16), stks[s])
              for s in (0, 1)]
        acc = [ys[s] + (dt / 6.0) * k1[s] for s in (0, 1)]
        k2 = [odefunc(t + 0.5 * dt,
                      (ys[s] + (0.5 * dt) * k1[s]).astype(jnp.bfloat16),
                      stks[s]) for s in (0, 1)]
        acc = [acc[s] + (dt / 3.0) * k2[s] for s in (0, 1)]
        k3 = [odefunc(t + 0.5 * dt,
                      (ys[s] + (0.5 * dt) * k2[s]).astype(jnp.bfloat16),
                      stks[s]) for s in (0, 1)]
        acc = [acc[s] + (dt / 3.0) * k3[s] for s in (0, 1)]
        k4 = [odefunc(t + dt,
                      (ys[s] + dt * k3[s]).astype(jnp.bfloat16),
                      stks[s]) for s in (0, 1)]
        return tuple(acc[s] + (dt / 6.0) * k4[s] for s in (0, 1))

    ya, yb = jax.lax.fori_loop(0, N_STEPS, rk4_step,
                               (x_ref[:, :nl2], x_ref[:, nl2:]))
    o_ref[:, :nl2] = ya
    o_ref[:, nl2:] = yb


def kernel(x, w1, b1, w2, b2):
    b = x.shape[0]
    nbs = NB if b % (2 * NB) == 0 else b // 2
    nl2 = nbs * W
    nl = 2 * nl2
    np_ = b // (2 * nbs)

    x = x.astype(jnp.float32)
    xp = (x.reshape(2 * np_, nbs, C, H, W)
           .transpose(2, 3, 0, 4, 1)
           .reshape(R, b * W))

    wv = jnp.arange(W)
    vw = jnp.stack([((wv + k - 1) >= 0) & ((wv + k - 1) < W)
                    for k in range(3)]).astype(jnp.float32)
    tw = jnp.concatenate(
        [jnp.repeat(vw, nbs, axis=1),
         jnp.ones((1, nl2), jnp.float32),
         jnp.zeros((4, nl2), jnp.float32)], axis=0).astype(jnp.bfloat16)

    hv = jnp.arange(H)
    vh = jnp.stack([((hv + k - 1) >= 0) & ((hv + k - 1) < H)
                    for k in range(3)]).astype(jnp.float32)
    m1t = jnp.einsum('ckl,kh->chl', w1[:, 0].astype(jnp.float32),
                     vh).reshape(R, 3)
    b1c = jnp.repeat(b1.astype(jnp.float32), H)[:, None]
    b2c = jnp.repeat(b2.astype(jnp.float32), H)[:, None]

    m1l, m1c, m1r = _band_weights(w1[:, 1:])
    m2l, m2c, m2r = _band_weights(w2)
    z4 = jnp.zeros((R, 4), jnp.float32)
    z3 = jnp.zeros((R, 3), jnp.float32)
    z8 = jnp.zeros((R, 8), jnp.float32)
    m1 = jnp.concatenate([m1c, m1t, b1c, z4, m1l, m1r, z8],
                         axis=1).astype(jnp.bfloat16)
    m2 = jnp.concatenate([m2c, z3, b2c, z4, m2l, m2r, z8],
                         axis=1).astype(jnp.bfloat16)

    scratch = pltpu.VMEM((RS, nl2 + 2 * nbs), jnp.bfloat16)
    out = pl.pallas_call(
        _ode_kernel,
        out_shape=jax.ShapeDtypeStruct((R, b * W), jnp.float32),
        grid=(np_,),
        in_specs=[
            pl.BlockSpec((R, nl), lambda p: (0, p)),
            pl.BlockSpec((R, RS), lambda p: (0, 0)),
            pl.BlockSpec((R, RS), lambda p: (0, 0)),
            pl.BlockSpec((8, nl2), lambda p: (0, 0)),
        ],
        out_specs=pl.BlockSpec((R, nl), lambda p: (0, p)),
        scratch_shapes=[scratch, scratch],
        compiler_params=pltpu.CompilerParams(
            dimension_semantics=("parallel",)),
    )(xp, m1, m2, tw)

    return (out.reshape(C, H, 2 * np_, W, nbs)
               .transpose(2, 4, 0, 1, 3)
               .reshape(b, C, H, W))
